# baseline (speedup 1.0000x reference)
import functools
import math

import numpy as np
import jax
import jax.numpy as jnp
from jax import lax
from jax.experimental import pallas as pl
from jax.experimental.pallas import tpu as pltpu
from jax.experimental.pallas import tpu_sc as plsc

_INTERPRET = False


def _dot(a, b, precision=lax.Precision.HIGHEST):
    return lax.dot_general(
        a, b, (((1,), (0,)), ((), ())),
        preferred_element_type=jnp.float32,
        precision=precision,
    )


_FAST = lax.Precision.DEFAULT


def _dotf(a, b):
    return lax.dot_general(
        a, b, (((1,), (0,)), ((), ())),
        preferred_element_type=jnp.float32,
        precision=lax.Precision.DEFAULT,
    )


def _bsplit(x):
    h = x.astype(jnp.bfloat16)
    l = (x - h.astype(jnp.float32)).astype(jnp.bfloat16)
    return h, l


def _dot3(ah, al, bh, bl):
    return _dotf(ah, bh) + _dotf(ah, bl) + _dotf(al, bh)


_FCHUNK = 256


def _tc1_body(k_sel, F, L, dblk, ach, acl, ash, asl, bch, bcl, bsh, bsl,
              q, k, v, w_out, i_out, cv_out, sv_out):
    Qh, Ql = _bsplit(q[0])
    Kh, Kl = _bsplit(k[0])
    Vh = v[0].astype(jnp.bfloat16)
    rez, imz = [], []
    for fi in range(F // _FCHUNK):
        fs = pl.ds(fi * _FCHUNK, _FCHUNK)
        ch = ach[fs, :]
        cl = acl[fs, :]
        sh = ash[fs, :]
        sl = asl[fs, :]
        CQ = _dot3(ch, cl, Qh, Ql)
        SQ = _dot3(sh, sl, Qh, Ql)
        CK = _dot3(ch, cl, Kh, Kl)
        SK = _dot3(sh, sl, Kh, Kl)
        cv_out[0, fs, :] = _dotf(ch, Vh)
        sv_out[0, fs, :] = _dotf(sh, Vh)
        rez.append(CQ * CK + SQ * SK)
        imz.append(CQ * SK - SQ * CK)
    Zh, Zl = _bsplit(jnp.concatenate(rez, axis=0))
    Yh, Yl = _bsplit(jnp.concatenate(imz, axis=0))
    r = (_dot3(bch[...], bcl[...], Zh, Zl)
         - _dot3(bsh[...], bsl[...], Yh, Yl))
    iota_l = lax.broadcasted_iota(jnp.int32, (L, dblk), 0)
    ws, idxs = [], []
    for _ in range(k_sel):
        m = jnp.max(r, axis=0)
        hit = r == m[None, :]
        idx = jnp.min(jnp.where(hit, iota_l, L), axis=0)
        ws.append(m)
        idxs.append(idx)
        r = jnp.where(iota_l == idx[None, :], -jnp.inf, r)
    Wm = jnp.stack(ws)
    mx = jnp.max(Wm, axis=0, keepdims=True)
    e = jnp.exp(Wm - mx)
    Wm = e / jnp.sum(e, axis=0, keepdims=True)
    pad = 16 - k_sel
    w_out[0] = jnp.concatenate(
        [Wm, jnp.zeros((pad, dblk), jnp.float32)], axis=0)
    i_out[0] = jnp.concatenate(
        [jnp.stack(idxs), jnp.zeros((pad, dblk), jnp.int32)], axis=0)


def _tc2_body(F, L, acT, asT, bc, bs, wf, cv, sv, v, a_out):
    Wfb = wf[0]
    rey, imy = [], []
    for fi in range(F // _FCHUNK):
        fs = pl.ds(fi * _FCHUNK, _FCHUNK)
        Cw = _dotf(acT[fs, :], Wfb)
        Sw = _dotf(asT[fs, :], Wfb)
        CVb = cv[0, fs, :]
        SVb = sv[0, fs, :]
        rey.append(Cw * CVb + Sw * SVb)
        imy.append(Sw * CVb - Cw * SVb)
    ReY = jnp.concatenate(rey, axis=0)
    ImY = jnp.concatenate(imy, axis=0)
    vmean = jnp.sum(v[0], axis=0, keepdims=True) * (1.0 / L)
    for li in range(L // _FCHUNK):
        ls = pl.ds(li * _FCHUNK, _FCHUNK)
        a_out[0, ls, :] = (_dotf(bc[ls, :], ReY)
                           - _dotf(bs[ls, :], ImY) + vmean)


def _sc_scatter_body(k_sel, L, d, w_hbm, i_hbm, z_hbm, out_hbm, wv, iv, buf):
    nc = 2
    nlw = 16
    lwin = L // nlw
    wid = lax.axis_index("s") * nc + lax.axis_index("c")
    b = wid // nlw
    l0 = (wid % nlw) * lwin
    nkd = 16 * d
    pltpu.sync_copy(w_hbm.at[pl.ds(pl.multiple_of(b * nkd, 8), nkd)], wv)
    pltpu.sync_copy(i_hbm.at[pl.ds(pl.multiple_of(b * nkd, 8), nkd)], iv)
    ng = d // 16
    nwords = lwin * d
    pltpu.sync_copy(z_hbm, buf)

    def _scatter_j(j, carry):
        jbase = pl.multiple_of(j * d, 8)
        for g in range(ng):
            rows = iv[pl.ds(jbase + g * 16, 16)] - l0
            vals = wv[pl.ds(jbase + g * 16, 16)]
            mask = (rows >= 0) & (rows < lwin)
            rows_c = jnp.minimum(jnp.maximum(rows, 0), lwin - 1)
            cols = lax.iota(jnp.int32, 16) + (g * 16)
            plsc.store_scatter(buf, [rows_c * d + cols], vals, mask=mask)
        return carry

    lax.fori_loop(0, k_sel, _scatter_j, 0)
    pltpu.sync_copy(
        buf, out_hbm.at[pl.ds(pl.multiple_of(b * (L * d) + l0 * d, 8), nwords)])


def kernel(Q, K, V):
    B, L, d = Q.shape
    F = L // 2
    k_sel = min(int(math.floor(2 * math.log(L))), L)
    dblk = 128
    nd = d // dblk
    dblk2 = 256
    nd2 = d // dblk2

    fr = np.arange(1, F + 1, dtype=np.float64)
    ll = np.arange(L, dtype=np.float64)
    theta = 2.0 * np.pi * np.outer(fr, ll) / L
    cosv = np.cos(theta)
    sinv = np.sin(theta)
    AcT = cosv.astype(np.float32)
    AsT = sinv.astype(np.float32)
    wgt = (np.where(fr == F, 1.0, 2.0) / L)[:, None]
    Bc = np.ascontiguousarray((cosv * wgt).T).astype(np.float32)
    Bs = np.ascontiguousarray((sinv * wgt).T).astype(np.float32)

    def _host_split(x):
        h = x.astype(np.float32).astype(jnp.bfloat16)
        l = jnp.asarray(x - np.asarray(h, np.float32), jnp.bfloat16)
        return h, l

    AcTh, AcTl = _host_split(AcT)
    AsTh, AsTl = _host_split(AsT)
    Bch, Bcl = _host_split(Bc)
    Bsh, Bsl = _host_split(Bs)

    const_spec2 = lambda shape: pl.BlockSpec(shape, lambda b, n: (0, 0))
    blk3 = lambda rows: pl.BlockSpec((1, rows, dblk), lambda b, n: (b, 0, n))
    blk3b = lambda rows: pl.BlockSpec((1, rows, dblk2), lambda b, n: (b, 0, n))

    tc1 = pl.pallas_call(
        functools.partial(_tc1_body, k_sel, F, L, dblk),
        grid=(B, nd),
        in_specs=[const_spec2((F, L))] * 4 + [const_spec2((L, F))] * 4
        + [blk3(L), blk3(L), blk3(L)],
        out_specs=[blk3(16), blk3(16), blk3(F), blk3(F)],
        out_shape=[
            jax.ShapeDtypeStruct((B, 16, d), jnp.float32),
            jax.ShapeDtypeStruct((B, 16, d), jnp.int32),
            jax.ShapeDtypeStruct((B, F, d), jnp.float32),
            jax.ShapeDtypeStruct((B, F, d), jnp.float32),
        ],
        interpret=_INTERPRET,
    )
    Wk, Ik, CV, SV = tc1(AcTh, AcTl, AsTh, AsTl, Bch, Bcl, Bsh, Bsl, Q, K, V)

    wfull = _sc_scatter(Wk, Ik, B, L, d, k_sel)

    tc2 = pl.pallas_call(
        functools.partial(_tc2_body, F, L),
        grid=(B, nd2),
        in_specs=[const_spec2((F, L)), const_spec2((F, L)),
                  const_spec2((L, F)), const_spec2((L, F)),
                  blk3b(L), blk3b(F), blk3b(F), blk3b(L)],
        out_specs=blk3b(L),
        out_shape=jax.ShapeDtypeStruct((B, L, d), jnp.float32),
        interpret=_INTERPRET,
    )
    return tc2(AcT, AsT, Bc, Bs, wfull, CV, SV, V)


def _sc_scatter(Wk, Ik, B, L, d, k_sel):
    mesh = plsc.VectorSubcoreMesh(core_axis_name="c", subcore_axis_name="s")
    sc = pl.kernel(
        functools.partial(_sc_scatter_body, k_sel, L, d),
        mesh=mesh,
        compiler_params=pltpu.CompilerParams(needs_layout_passes=False),
        out_type=jax.ShapeDtypeStruct((B * L * d,), jnp.float32),
        scratch_types=[
            pltpu.VMEM((16 * d,), jnp.float32),
            pltpu.VMEM((16 * d,), jnp.int32),
            pltpu.VMEM(((L // 16) * d,), jnp.float32),
        ],
    )
    zblk = jnp.zeros(((L // 16) * d,), jnp.float32)
    return sc(Wk.reshape(B * 16 * d), Ik.reshape(B * 16 * d),
              zblk).reshape(B, L, d)


# speedup vs baseline: 1.0140x; 1.0140x over previous
import functools
import math

import numpy as np
import jax
import jax.numpy as jnp
from jax import lax
from jax.experimental import pallas as pl
from jax.experimental.pallas import tpu as pltpu
from jax.experimental.pallas import tpu_sc as plsc

def _dot(a, b, precision=lax.Precision.HIGHEST):
    return lax.dot_general(
        a, b, (((1,), (0,)), ((), ())),
        preferred_element_type=jnp.float32,
        precision=precision,
    )


_FAST = lax.Precision.DEFAULT


def _dotf(a, b):
    return lax.dot_general(
        a, b, (((1,), (0,)), ((), ())),
        preferred_element_type=jnp.float32,
        precision=lax.Precision.DEFAULT,
    )


def _bsplit(x):
    h = x.astype(jnp.bfloat16)
    l = (x - h.astype(jnp.float32)).astype(jnp.bfloat16)
    return h, l


def _dot3(ah, al, bh, bl):
    return _dotf(ah, bh) + _dotf(ah, bl) + _dotf(al, bh)


_FCHUNK = 256


def _tc1_body(k_sel, F, L, dblk, ach, acl, ash, asl, bch, bcl, bsh, bsl,
              q, k, v, w_out, i_out, cv_out, sv_out):
    Qh, Ql = _bsplit(q[0])
    Kh, Kl = _bsplit(k[0])
    Vh = v[0].astype(jnp.bfloat16)
    rez, imz = [], []
    for fi in range(F // _FCHUNK):
        fs = pl.ds(fi * _FCHUNK, _FCHUNK)
        ch = ach[fs, :]
        cl = acl[fs, :]
        sh = ash[fs, :]
        sl = asl[fs, :]
        CQ = _dot3(ch, cl, Qh, Ql)
        SQ = _dot3(sh, sl, Qh, Ql)
        CK = _dot3(ch, cl, Kh, Kl)
        SK = _dot3(sh, sl, Kh, Kl)
        cv_out[0, fs, :] = _dotf(ch, Vh)
        sv_out[0, fs, :] = _dotf(sh, Vh)
        rez.append(CQ * CK + SQ * SK)
        imz.append(CQ * SK - SQ * CK)
    Zh, Zl = _bsplit(jnp.concatenate(rez, axis=0))
    Yh, Yl = _bsplit(jnp.concatenate(imz, axis=0))
    r = (_dot3(bch[...], bcl[...], Zh, Zl)
         - _dot3(bsh[...], bsl[...], Yh, Yl))
    iota_l = lax.broadcasted_iota(jnp.int32, (L, dblk), 0)
    ws, idxs = [], []
    for _ in range(k_sel):
        m = jnp.max(r, axis=0)
        hit = r == m[None, :]
        idx = jnp.min(jnp.where(hit, iota_l, L), axis=0)
        ws.append(m)
        idxs.append(idx)
        r = jnp.where(iota_l == idx[None, :], -jnp.inf, r)
    Wm = jnp.stack(ws)
    mx = jnp.max(Wm, axis=0, keepdims=True)
    e = jnp.exp(Wm - mx)
    Wm = e / jnp.sum(e, axis=0, keepdims=True)
    pad = 16 - k_sel
    w_out[0] = jnp.concatenate(
        [Wm, jnp.zeros((pad, dblk), jnp.float32)], axis=0)
    i_out[0] = jnp.concatenate(
        [jnp.stack(idxs), jnp.zeros((pad, dblk), jnp.int32)], axis=0)


def _tc2_body(F, L, acT, asT, bc, bs, wf, cv, sv, v, a_out):
    Wfb = wf[0]
    rey, imy = [], []
    for fi in range(F // _FCHUNK):
        fs = pl.ds(fi * _FCHUNK, _FCHUNK)
        Cw = _dotf(acT[fs, :], Wfb)
        Sw = _dotf(asT[fs, :], Wfb)
        CVb = cv[0, fs, :]
        SVb = sv[0, fs, :]
        rey.append(Cw * CVb + Sw * SVb)
        imy.append(Sw * CVb - Cw * SVb)
    ReY = jnp.concatenate(rey, axis=0)
    ImY = jnp.concatenate(imy, axis=0)
    vmean = jnp.sum(v[0], axis=0, keepdims=True) * (1.0 / L)
    for li in range(L // _FCHUNK):
        ls = pl.ds(li * _FCHUNK, _FCHUNK)
        a_out[0, ls, :] = (_dotf(bc[ls, :], ReY)
                           - _dotf(bs[ls, :], ImY) + vmean)


def _sc_scatter_body(k_sel, L, d, w_hbm, i_hbm, out_hbm, wv, iv, buf):
    nc = 2
    nlw = 16
    lwin = L // nlw
    wid = lax.axis_index("s") * nc + lax.axis_index("c")
    b = wid // nlw
    l0 = (wid % nlw) * lwin
    nkd = 16 * d
    pltpu.sync_copy(w_hbm.at[pl.ds(pl.multiple_of(b * nkd, 8), nkd)], wv)
    pltpu.sync_copy(i_hbm.at[pl.ds(pl.multiple_of(b * nkd, 8), nkd)], iv)
    zeros16 = jnp.zeros((16,), jnp.float32)
    ng = d // 16
    nwords = lwin * d

    def _zero_blk(i, carry):
        base = pl.multiple_of(i * 256, 256)
        for t in range(16):
            buf[pl.ds(base + t * 16, 16)] = zeros16
        return carry

    lax.fori_loop(0, nwords // 256, _zero_blk, 0)

    def _scatter_j(j, carry):
        jbase = pl.multiple_of(j * d, 8)
        for g in range(ng):
            rows = iv[pl.ds(jbase + g * 16, 16)] - l0
            vals = wv[pl.ds(jbase + g * 16, 16)]
            mask = (rows >= 0) & (rows < lwin)
            rows_c = jnp.minimum(jnp.maximum(rows, 0), lwin - 1)
            cols = lax.iota(jnp.int32, 16) + (g * 16)
            plsc.store_scatter(buf, [rows_c * d + cols], vals, mask=mask)
        return carry

    lax.fori_loop(0, k_sel, _scatter_j, 0)
    pltpu.sync_copy(
        buf, out_hbm.at[pl.ds(pl.multiple_of(b * (L * d) + l0 * d, 8), nwords)])


def kernel(Q, K, V):
    B, L, d = Q.shape
    F = L // 2
    k_sel = min(int(math.floor(2 * math.log(L))), L)
    dblk = 128
    nd = d // dblk
    dblk2 = 256
    nd2 = d // dblk2

    fr = np.arange(1, F + 1, dtype=np.float64)
    ll = np.arange(L, dtype=np.float64)
    theta = 2.0 * np.pi * np.outer(fr, ll) / L
    cosv = np.cos(theta)
    sinv = np.sin(theta)
    AcT = cosv.astype(np.float32)
    AsT = sinv.astype(np.float32)
    wgt = (np.where(fr == F, 1.0, 2.0) / L)[:, None]
    Bc = np.ascontiguousarray((cosv * wgt).T).astype(np.float32)
    Bs = np.ascontiguousarray((sinv * wgt).T).astype(np.float32)

    def _host_split(x):
        h = x.astype(np.float32).astype(jnp.bfloat16)
        l = jnp.asarray(x - np.asarray(h, np.float32), jnp.bfloat16)
        return h, l

    AcTh, AcTl = _host_split(AcT)
    AsTh, AsTl = _host_split(AsT)
    Bch, Bcl = _host_split(Bc)
    Bsh, Bsl = _host_split(Bs)

    const_spec2 = lambda shape: pl.BlockSpec(shape, lambda b, n: (0, 0))
    blk3 = lambda rows: pl.BlockSpec((1, rows, dblk), lambda b, n: (b, 0, n))
    blk3b = lambda rows: pl.BlockSpec((1, rows, dblk2), lambda b, n: (b, 0, n))

    tc1 = pl.pallas_call(
        functools.partial(_tc1_body, k_sel, F, L, dblk),
        grid=(B, nd),
        in_specs=[const_spec2((F, L))] * 4 + [const_spec2((L, F))] * 4
        + [blk3(L), blk3(L), blk3(L)],
        out_specs=[blk3(16), blk3(16), blk3(F), blk3(F)],
        out_shape=[
            jax.ShapeDtypeStruct((B, 16, d), jnp.float32),
            jax.ShapeDtypeStruct((B, 16, d), jnp.int32),
            jax.ShapeDtypeStruct((B, F, d), jnp.float32),
            jax.ShapeDtypeStruct((B, F, d), jnp.float32),
        ],
    )
    Wk, Ik, CV, SV = tc1(AcTh, AcTl, AsTh, AsTl, Bch, Bcl, Bsh, Bsl, Q, K, V)

    wfull = _sc_scatter(Wk, Ik, B, L, d, k_sel)

    tc2 = pl.pallas_call(
        functools.partial(_tc2_body, F, L),
        grid=(B, nd2),
        in_specs=[const_spec2((F, L)), const_spec2((F, L)),
                  const_spec2((L, F)), const_spec2((L, F)),
                  blk3b(L), blk3b(F), blk3b(F), blk3b(L)],
        out_specs=blk3b(L),
        out_shape=jax.ShapeDtypeStruct((B, L, d), jnp.float32),
    )
    return tc2(AcT, AsT, Bc, Bs, wfull, CV, SV, V)


def _sc_scatter(Wk, Ik, B, L, d, k_sel):
    mesh = plsc.VectorSubcoreMesh(core_axis_name="c", subcore_axis_name="s")
    sc = pl.kernel(
        functools.partial(_sc_scatter_body, k_sel, L, d),
        mesh=mesh,
        compiler_params=pltpu.CompilerParams(needs_layout_passes=False),
        out_type=jax.ShapeDtypeStruct((B * L * d,), jnp.float32),
        scratch_types=[
            pltpu.VMEM((16 * d,), jnp.float32),
            pltpu.VMEM((16 * d,), jnp.int32),
            pltpu.VMEM(((L // 16) * d,), jnp.float32),
        ],
    )
    return sc(Wk.reshape(B * 16 * d), Ik.reshape(B * 16 * d)).reshape(B, L, d)
